# baseline (speedup 1.0000x reference)
import functools

import jax
import jax.numpy as jnp
from jax import lax
from jax.experimental import pallas as pl
from jax.experimental.pallas import tpu as pltpu
from jax.experimental.pallas import tpu_sc as plsc

_STRIDE = 16
_MIN_CONF = 0.1
_B, _H, _W = 8, 14, 14
_HS, _WS = _H * _STRIDE, _W * _STRIDE
_N = _H * _W
_NP = 224
_WIN = 48
_R = _WIN // 2
_XPAD = 32
_SW = _WS + 2 * _XPAD
_KSC = 1
_NSTRIP = 32 // _KSC
_SH = _HS // _NSTRIP
_STRIP_WORDS = _SH * _SW
_NROW = 7


def _sc_body(params_hbm, out_hbm, pvm, gxtab, wgytab, cfx, cfy, chv, cce,
             cx0, cy0, cjl, cjh, strip):
    lanes = lax.iota(jnp.int32, 16)
    wid = lax.axis_index("s") * 2 + lax.axis_index("c")
    b = wid // _NSTRIP
    ys0 = (wid % _NSTRIP) * _SH

    pltpu.sync_copy(params_hbm.at[b], pvm)

    zeros = jnp.zeros((16,), jnp.float32)

    def zero_body(i, _):
        strip[pl.ds(i * 16, 16)] = zeros
        return _
    lax.fori_loop(0, _STRIP_WORDS // 16, zero_body, None, unroll=8)

    cnt = jnp.int32(0)
    for g in range(_NP // 16):
        mxg = pvm[pl.ds(0 * _NP + g * 16, 16)]
        myg = pvm[pl.ds(1 * _NP + g * 16, 16)]
        vg = pvm[pl.ds(2 * _NP + g * 16, 16)]
        cg = pvm[pl.ds(3 * _NP + g * 16, 16)]
        jl0 = pvm[pl.ds(4 * _NP + g * 16, 16)].astype(jnp.int32)
        jh0 = pvm[pl.ds(5 * _NP + g * 16, 16)].astype(jnp.int32)
        hv = 0.5 / vg
        x0i = mxg.astype(jnp.int32)
        y0i = myg.astype(jnp.int32)
        fx = mxg - x0i.astype(jnp.float32) + float(_R)
        fy = myg - y0i.astype(jnp.float32) + float(_R)
        y0w = y0i - _R
        mask = ((cg > _MIN_CONF)
                & (y0w + jl0 < ys0 + _SH)
                & (y0w + jh0 > ys0))
        pos = cnt + lax.cumsum(mask.astype(jnp.int32), axis=0) - 1
        plsc.store_scatter(cfx, [pos], fx, mask=mask)
        plsc.store_scatter(cfy, [pos], fy, mask=mask)
        plsc.store_scatter(chv, [pos], hv, mask=mask)
        plsc.store_scatter(cce, [pos], cg, mask=mask)
        plsc.store_scatter(cx0, [pos], x0i - _R + _XPAD, mask=mask)
        plsc.store_scatter(cy0, [pos], y0w, mask=mask)
        plsc.store_scatter(cjl, [pos], jl0, mask=mask)
        plsc.store_scatter(cjh, [pos], jh0, mask=mask)
        cnt = cnt + plsc.all_reduce_population_count(mask)[0]

    def tabgrp_body(q, _):
        fx = cfx[pl.ds(q * 16, 16)]
        fy = cfy[pl.ds(q * 16, 16)]
        hv = chv[pl.ds(q * 16, 16)]
        ce = cce[pl.ds(q * 16, 16)]
        idx0 = (q * 16 + lanes) * _WIN

        def tab_body(j, __):
            jf = j.astype(jnp.float32)
            dx = jf - fx
            gx = jnp.exp(-(dx * dx) * hv)
            dy = jf - fy
            gy = jnp.exp(-(dy * dy) * hv) * ce
            plsc.store_scatter(gxtab, [idx0 + j], gx)
            plsc.store_scatter(wgytab, [idx0 + j], gy)
            return __
        lax.fori_loop(0, _WIN, tab_body, None, unroll=2)
        return _
    lax.fori_loop(0, (cnt + 15) // 16, tabgrp_body, None)

    def cell_body(n, _):
        x0 = cx0[pl.ds(n, 16)][0]
        y0 = cy0[pl.ds(n, 16)][0]
        jlo = jnp.maximum(cjl[pl.ds(n, 16)][0], ys0 - y0)
        jhi = jnp.minimum(cjh[pl.ds(n, 16)][0], ys0 + _SH - y0)
        tbase = n * _WIN
        gx0 = plsc.load_gather(gxtab, [tbase + lanes])
        gx1 = plsc.load_gather(gxtab, [tbase + 16 + lanes])
        gx2 = plsc.load_gather(gxtab, [tbase + 32 + lanes])
        i0_init = (y0 + jlo - ys0) * _SW + x0 + lanes
        sbase = tbase + jlo

        def row_body(t, i0):
            s = wgytab[pl.ds(sbase + t, 16)][0]
            plsc.addupdate_scatter(strip, [i0], gx0 * s)
            plsc.addupdate_scatter(strip, [i0 + 16], gx1 * s)
            plsc.addupdate_scatter(strip, [i0 + 32], gx2 * s)
            return i0 + _SW
        lax.fori_loop(0, jnp.maximum(0, jhi - jlo), row_body, i0_init)
        return _
    lax.fori_loop(0, cnt, cell_body, None)

    pltpu.sync_copy(strip, out_hbm.at[pl.ds(wid * _STRIP_WORDS, _STRIP_WORDS)])


def _tc_body(my_ref, vr_ref, mx_ref, vc_ref, c_ref, out_ref):
    hs, np_ = out_ref.shape[1], my_ref.shape[2]
    ws = out_ref.shape[2]
    my = my_ref[0]
    hv_r = 0.5 / vr_ref[0]
    mx = mx_ref[0]
    hv_c = 0.5 / vc_ref[0]
    c = c_ref[0]
    ceff = jnp.where(c > _MIN_CONF, c, 0.0)

    y = lax.broadcasted_iota(jnp.int32, (hs, np_), 0).astype(jnp.float32)
    dy = y - my
    gyt = jnp.exp(-(dy * dy) * hv_r)

    x = lax.broadcasted_iota(jnp.int32, (np_, ws), 1).astype(jnp.float32)
    dx = x - mx
    cgx = ceff * jnp.exp(-(dx * dx) * hv_c)

    out_ref[0] = jnp.dot(gyt, cgx, preferred_element_type=jnp.float32)


def _tc_part(mean, variance, confidence):
    nb = mean.shape[0]
    m = mean.reshape(nb, _N, 2)
    pad = _NP - _N
    mx = jnp.pad(m[..., 0], ((0, 0), (0, pad)))
    my = jnp.pad(m[..., 1], ((0, 0), (0, pad)))
    v = jnp.pad(variance.reshape(nb, _N), ((0, 0), (0, pad)),
                constant_values=1.0)
    c = jnp.pad(confidence.reshape(nb, _N), ((0, 0), (0, pad)))

    row_spec = pl.BlockSpec((1, 1, _NP), lambda i: (i, 0, 0))
    col_spec = pl.BlockSpec((1, _NP, 1), lambda i: (i, 0, 0))
    return pl.pallas_call(
        _tc_body,
        grid=(nb,),
        in_specs=[row_spec, row_spec, col_spec, col_spec, col_spec],
        out_specs=pl.BlockSpec((1, _HS, _WS), lambda i: (i, 0, 0)),
        out_shape=jax.ShapeDtypeStruct((nb, _HS, _WS), jnp.float32),
    )(my[:, None, :], v[:, None, :], mx[:, :, None], v[:, :, None],
      c[:, :, None])


def _sc_part(mean, variance, confidence):
    nb = mean.shape[0]
    m = mean.reshape(nb, _N, 2)
    pad = _NP - _N
    mx = jnp.pad(m[..., 0], ((0, 0), (0, pad)), constant_values=112.0)
    my = jnp.pad(m[..., 1], ((0, 0), (0, pad)), constant_values=112.0)
    v = jnp.pad(variance.reshape(nb, _N), ((0, 0), (0, pad)),
                constant_values=1.0)
    c = jnp.pad(confidence.reshape(nb, _N), ((0, 0), (0, pad)))
    r = jnp.ceil(jnp.sqrt(2.0 * jnp.log(1.0 / 3e-4) * v))
    alive = c > _MIN_CONF
    jl0 = jnp.where(alive, jnp.maximum(0.0, float(_R) - r), 0.0)
    jh0 = jnp.where(alive, jnp.minimum(float(_WIN), float(_R) + r + 2.0), 0.0)
    params = jnp.stack([mx, my, v, c, jl0, jh0, jnp.zeros_like(mx)],
                       axis=1).reshape(nb, _NROW * _NP)

    mesh = plsc.VectorSubcoreMesh(core_axis_name="c", subcore_axis_name="s")
    run = functools.partial(
        pl.kernel,
        mesh=mesh,
        compiler_params=pltpu.CompilerParams(needs_layout_passes=False),
        out_type=jax.ShapeDtypeStruct((nb * _HS * _SW,), jnp.float32),
        scratch_types=[
            pltpu.VMEM((_NROW * _NP,), jnp.float32),
            pltpu.VMEM((_NP * _WIN,), jnp.float32),
            pltpu.VMEM((_NP * _WIN + 16,), jnp.float32),
            pltpu.VMEM((_NP + 16,), jnp.float32),
            pltpu.VMEM((_NP + 16,), jnp.float32),
            pltpu.VMEM((_NP + 16,), jnp.float32),
            pltpu.VMEM((_NP + 16,), jnp.float32),
            pltpu.VMEM((_NP + 16,), jnp.int32),
            pltpu.VMEM((_NP + 16,), jnp.int32),
            pltpu.VMEM((_NP + 16,), jnp.int32),
            pltpu.VMEM((_NP + 16,), jnp.int32),
            pltpu.VMEM((_STRIP_WORDS,), jnp.float32),
        ],
    )(_sc_body)
    padded = run(params)
    return padded.reshape(nb, _HS, _SW)[:, :, _XPAD:_XPAD + _WS]


def kernel(mean, variance, confidence):
    sc = _sc_part(mean[:_KSC], variance[:_KSC], confidence[:_KSC])
    tc = _tc_part(mean[_KSC:], variance[_KSC:], confidence[_KSC:])
    return jnp.concatenate([sc, tc], axis=0)


# speedup vs baseline: 1.0405x; 1.0405x over previous
import functools

import jax
import jax.numpy as jnp
from jax import lax
from jax.experimental import pallas as pl
from jax.experimental.pallas import tpu as pltpu
from jax.experimental.pallas import tpu_sc as plsc

_STRIDE = 16
_MIN_CONF = 0.1
_B, _H, _W = 8, 14, 14
_HS, _WS = _H * _STRIDE, _W * _STRIDE
_N = _H * _W
_NP = 224
_WIN = 48
_R = _WIN // 2
_XPAD = 32
_SW = _WS + 2 * _XPAD
_KSC = 4
_NSTRIP = 32 // _KSC
_SH = _HS // _NSTRIP
_STRIP_WORDS = _SH * _SW
_NROW = 7


def _sc_body(params_hbm, out_hbm, pvm, gxtab, wgytab, cfx, cfy, chv, cce,
             cx0, cy0, cjl, cjh, strip):
    lanes = lax.iota(jnp.int32, 16)
    wid = lax.axis_index("s") * 2 + lax.axis_index("c")
    b = wid // _NSTRIP
    ys0 = (wid % _NSTRIP) * _SH

    pltpu.sync_copy(params_hbm.at[b], pvm)

    zeros = jnp.zeros((16,), jnp.float32)

    def zero_body(i, _):
        strip[pl.ds(i * 16, 16)] = zeros
        return _
    lax.fori_loop(0, _STRIP_WORDS // 16, zero_body, None, unroll=8)

    cnt = jnp.int32(0)
    for g in range(_NP // 16):
        mxg = pvm[pl.ds(0 * _NP + g * 16, 16)]
        myg = pvm[pl.ds(1 * _NP + g * 16, 16)]
        vg = pvm[pl.ds(2 * _NP + g * 16, 16)]
        cg = pvm[pl.ds(3 * _NP + g * 16, 16)]
        jl0 = pvm[pl.ds(4 * _NP + g * 16, 16)].astype(jnp.int32)
        jh0 = pvm[pl.ds(5 * _NP + g * 16, 16)].astype(jnp.int32)
        hv = 0.5 / vg
        x0i = mxg.astype(jnp.int32)
        y0i = myg.astype(jnp.int32)
        fx = mxg - x0i.astype(jnp.float32) + float(_R)
        fy = myg - y0i.astype(jnp.float32) + float(_R)
        y0w = y0i - _R
        mask = ((cg > _MIN_CONF)
                & (y0w + jl0 < ys0 + _SH)
                & (y0w + jh0 > ys0))
        pos = cnt + lax.cumsum(mask.astype(jnp.int32), axis=0) - 1
        plsc.store_scatter(cfx, [pos], fx, mask=mask)
        plsc.store_scatter(cfy, [pos], fy, mask=mask)
        plsc.store_scatter(chv, [pos], hv, mask=mask)
        plsc.store_scatter(cce, [pos], cg, mask=mask)
        plsc.store_scatter(cx0, [pos], x0i - _R + _XPAD, mask=mask)
        plsc.store_scatter(cy0, [pos], y0w, mask=mask)
        plsc.store_scatter(cjl, [pos], jl0, mask=mask)
        plsc.store_scatter(cjh, [pos], jh0, mask=mask)
        cnt = cnt + plsc.all_reduce_population_count(mask)[0]

    def tabgrp_body(q, _):
        fx = cfx[pl.ds(q * 16, 16)]
        fy = cfy[pl.ds(q * 16, 16)]
        hv = chv[pl.ds(q * 16, 16)]
        ce = cce[pl.ds(q * 16, 16)]
        idx0 = (q * 16 + lanes) * _WIN

        def tab_body(j, __):
            jf = j.astype(jnp.float32)
            dx = jf - fx
            gx = jnp.exp(-(dx * dx) * hv)
            dy = jf - fy
            gy = jnp.exp(-(dy * dy) * hv) * ce
            plsc.store_scatter(gxtab, [idx0 + j], gx)
            plsc.store_scatter(wgytab, [idx0 + j], gy)
            return __
        lax.fori_loop(0, _WIN, tab_body, None, unroll=2)
        return _
    lax.fori_loop(0, (cnt + 15) // 16, tabgrp_body, None)

    def cell_body(n, _):
        x0 = cx0[pl.ds(n, 16)][0]
        y0 = cy0[pl.ds(n, 16)][0]
        jlo = jnp.maximum(cjl[pl.ds(n, 16)][0], ys0 - y0)
        jhi = jnp.minimum(cjh[pl.ds(n, 16)][0], ys0 + _SH - y0)
        tbase = n * _WIN
        gx0 = plsc.load_gather(gxtab, [tbase + lanes])
        gx1 = plsc.load_gather(gxtab, [tbase + 16 + lanes])
        gx2 = plsc.load_gather(gxtab, [tbase + 32 + lanes])
        i0_init = (y0 + jlo - ys0) * _SW + x0 + lanes
        sbase = tbase + jlo

        def row_body(t, i0):
            s = wgytab[pl.ds(sbase + t, 16)][0]
            plsc.addupdate_scatter(strip, [i0], gx0 * s)
            plsc.addupdate_scatter(strip, [i0 + 16], gx1 * s)
            plsc.addupdate_scatter(strip, [i0 + 32], gx2 * s)
            return i0 + _SW
        lax.fori_loop(0, jnp.maximum(0, jhi - jlo), row_body, i0_init)
        return _
    lax.fori_loop(0, cnt, cell_body, None)

    pltpu.sync_copy(strip, out_hbm.at[pl.ds(wid * _STRIP_WORDS, _STRIP_WORDS)])


def _tc_body(my_ref, vr_ref, mx_ref, vc_ref, c_ref, out_ref):
    hs, np_ = out_ref.shape[1], my_ref.shape[2]
    ws = out_ref.shape[2]
    my = my_ref[0]
    hv_r = 0.5 / vr_ref[0]
    mx = mx_ref[0]
    hv_c = 0.5 / vc_ref[0]
    c = c_ref[0]
    ceff = jnp.where(c > _MIN_CONF, c, 0.0)

    y = lax.broadcasted_iota(jnp.int32, (hs, np_), 0).astype(jnp.float32)
    dy = y - my
    gyt = jnp.exp(-(dy * dy) * hv_r)

    x = lax.broadcasted_iota(jnp.int32, (np_, ws), 1).astype(jnp.float32)
    dx = x - mx
    cgx = ceff * jnp.exp(-(dx * dx) * hv_c)

    out_ref[0] = jnp.dot(gyt, cgx, preferred_element_type=jnp.float32)


def _tc_part(mean, variance, confidence):
    nb = mean.shape[0]
    m = mean.reshape(nb, _N, 2)
    pad = _NP - _N
    mx = jnp.pad(m[..., 0], ((0, 0), (0, pad)))
    my = jnp.pad(m[..., 1], ((0, 0), (0, pad)))
    v = jnp.pad(variance.reshape(nb, _N), ((0, 0), (0, pad)),
                constant_values=1.0)
    c = jnp.pad(confidence.reshape(nb, _N), ((0, 0), (0, pad)))

    row_spec = pl.BlockSpec((1, 1, _NP), lambda i: (i, 0, 0))
    col_spec = pl.BlockSpec((1, _NP, 1), lambda i: (i, 0, 0))
    return pl.pallas_call(
        _tc_body,
        grid=(nb,),
        in_specs=[row_spec, row_spec, col_spec, col_spec, col_spec],
        out_specs=pl.BlockSpec((1, _HS, _WS), lambda i: (i, 0, 0)),
        out_shape=jax.ShapeDtypeStruct((nb, _HS, _WS), jnp.float32),
    )(my[:, None, :], v[:, None, :], mx[:, :, None], v[:, :, None],
      c[:, :, None])


def _sc_part(mean, variance, confidence):
    nb = mean.shape[0]
    m = mean.reshape(nb, _N, 2)
    pad = _NP - _N
    mx = jnp.pad(m[..., 0], ((0, 0), (0, pad)), constant_values=112.0)
    my = jnp.pad(m[..., 1], ((0, 0), (0, pad)), constant_values=112.0)
    v = jnp.pad(variance.reshape(nb, _N), ((0, 0), (0, pad)),
                constant_values=1.0)
    c = jnp.pad(confidence.reshape(nb, _N), ((0, 0), (0, pad)))
    r = jnp.ceil(jnp.sqrt(2.0 * jnp.log(1.0 / 3e-4) * v))
    alive = c > _MIN_CONF
    jl0 = jnp.where(alive, jnp.maximum(0.0, float(_R) - r), 0.0)
    jh0 = jnp.where(alive, jnp.minimum(float(_WIN), float(_R) + r + 2.0), 0.0)
    params = jnp.stack([mx, my, v, c, jl0, jh0, jnp.zeros_like(mx)],
                       axis=1).reshape(nb, _NROW * _NP)

    mesh = plsc.VectorSubcoreMesh(core_axis_name="c", subcore_axis_name="s")
    run = functools.partial(
        pl.kernel,
        mesh=mesh,
        compiler_params=pltpu.CompilerParams(needs_layout_passes=False),
        out_type=jax.ShapeDtypeStruct((nb * _HS * _SW,), jnp.float32),
        scratch_types=[
            pltpu.VMEM((_NROW * _NP,), jnp.float32),
            pltpu.VMEM((_NP * _WIN,), jnp.float32),
            pltpu.VMEM((_NP * _WIN + 16,), jnp.float32),
            pltpu.VMEM((_NP + 16,), jnp.float32),
            pltpu.VMEM((_NP + 16,), jnp.float32),
            pltpu.VMEM((_NP + 16,), jnp.float32),
            pltpu.VMEM((_NP + 16,), jnp.float32),
            pltpu.VMEM((_NP + 16,), jnp.int32),
            pltpu.VMEM((_NP + 16,), jnp.int32),
            pltpu.VMEM((_NP + 16,), jnp.int32),
            pltpu.VMEM((_NP + 16,), jnp.int32),
            pltpu.VMEM((_STRIP_WORDS,), jnp.float32),
        ],
    )(_sc_body)
    padded = run(params)
    return padded.reshape(nb, _HS, _SW)[:, :, _XPAD:_XPAD + _WS]


def kernel(mean, variance, confidence):
    sc = _sc_part(mean[:_KSC], variance[:_KSC], confidence[:_KSC])
    tc = _tc_part(mean[_KSC:], variance[_KSC:], confidence[_KSC:])
    return jnp.concatenate([sc, tc], axis=0)


# speedup vs baseline: 1.1478x; 1.1032x over previous
import functools

import jax
import jax.numpy as jnp
from jax import lax
from jax.experimental import pallas as pl
from jax.experimental.pallas import tpu as pltpu
from jax.experimental.pallas import tpu_sc as plsc

_STRIDE = 16
_MIN_CONF = 0.1
_B, _H, _W = 8, 14, 14
_HS, _WS = _H * _STRIDE, _W * _STRIDE
_N = _H * _W
_NP = 224
_WIN = 48
_R = _WIN // 2
_XPAD = 32
_SW = _WS + 2 * _XPAD
_KSC = 2
_NSTRIP = 32 // _KSC
_SH = _HS // _NSTRIP
_STRIP_WORDS = _SH * _SW
_NROW = 7


def _sc_body(params_hbm, out_hbm, pvm, gxtab, wgytab, cfx, cfy, chv, cce,
             cx0, cy0, cjl, cjh, strip):
    lanes = lax.iota(jnp.int32, 16)
    wid = lax.axis_index("s") * 2 + lax.axis_index("c")
    b = wid // _NSTRIP
    ys0 = (wid % _NSTRIP) * _SH

    pltpu.sync_copy(params_hbm.at[b], pvm)

    zeros = jnp.zeros((16,), jnp.float32)

    def zero_body(i, _):
        strip[pl.ds(i * 16, 16)] = zeros
        return _
    lax.fori_loop(0, _STRIP_WORDS // 16, zero_body, None, unroll=8)

    cnt = jnp.int32(0)
    for g in range(_NP // 16):
        mxg = pvm[pl.ds(0 * _NP + g * 16, 16)]
        myg = pvm[pl.ds(1 * _NP + g * 16, 16)]
        vg = pvm[pl.ds(2 * _NP + g * 16, 16)]
        cg = pvm[pl.ds(3 * _NP + g * 16, 16)]
        jl0 = pvm[pl.ds(4 * _NP + g * 16, 16)].astype(jnp.int32)
        jh0 = pvm[pl.ds(5 * _NP + g * 16, 16)].astype(jnp.int32)
        hv = 0.5 / vg
        x0i = mxg.astype(jnp.int32)
        y0i = myg.astype(jnp.int32)
        fx = mxg - x0i.astype(jnp.float32) + float(_R)
        fy = myg - y0i.astype(jnp.float32) + float(_R)
        y0w = y0i - _R
        mask = ((cg > _MIN_CONF)
                & (y0w + jl0 < ys0 + _SH)
                & (y0w + jh0 > ys0))
        pos = cnt + lax.cumsum(mask.astype(jnp.int32), axis=0) - 1
        plsc.store_scatter(cfx, [pos], fx, mask=mask)
        plsc.store_scatter(cfy, [pos], fy, mask=mask)
        plsc.store_scatter(chv, [pos], hv, mask=mask)
        plsc.store_scatter(cce, [pos], cg, mask=mask)
        plsc.store_scatter(cx0, [pos], x0i - _R + _XPAD, mask=mask)
        plsc.store_scatter(cy0, [pos], y0w, mask=mask)
        plsc.store_scatter(cjl, [pos], jl0, mask=mask)
        plsc.store_scatter(cjh, [pos], jh0, mask=mask)
        cnt = cnt + plsc.all_reduce_population_count(mask)[0]

    def tabgrp_body(q, _):
        fx = cfx[pl.ds(q * 16, 16)]
        fy = cfy[pl.ds(q * 16, 16)]
        hv = chv[pl.ds(q * 16, 16)]
        ce = cce[pl.ds(q * 16, 16)]
        idx0 = (q * 16 + lanes) * _WIN

        def tab_body(j, __):
            jf = j.astype(jnp.float32)
            dx = jf - fx
            gx = jnp.exp(-(dx * dx) * hv)
            dy = jf - fy
            gy = jnp.exp(-(dy * dy) * hv) * ce
            plsc.store_scatter(gxtab, [idx0 + j], gx)
            plsc.store_scatter(wgytab, [idx0 + j], gy)
            return __
        lax.fori_loop(0, _WIN, tab_body, None, unroll=2)
        return _
    lax.fori_loop(0, (cnt + 15) // 16, tabgrp_body, None)

    def cell_body(n, _):
        x0 = cx0[pl.ds(n, 16)][0]
        y0 = cy0[pl.ds(n, 16)][0]
        jlo = jnp.maximum(cjl[pl.ds(n, 16)][0], ys0 - y0)
        jhi = jnp.minimum(cjh[pl.ds(n, 16)][0], ys0 + _SH - y0)
        tbase = n * _WIN
        gx0 = plsc.load_gather(gxtab, [tbase + lanes])
        gx1 = plsc.load_gather(gxtab, [tbase + 16 + lanes])
        gx2 = plsc.load_gather(gxtab, [tbase + 32 + lanes])
        i0_init = (y0 + jlo - ys0) * _SW + x0 + lanes
        sbase = tbase + jlo

        nrows = jnp.maximum(0, jhi - jlo)
        npairs = nrows // 2

        def pair_body(t, i0):
            sv = wgytab[pl.ds(sbase + 2 * t, 16)]
            s0 = sv[0]
            s1 = sv[1]
            plsc.addupdate_scatter(strip, [i0], gx0 * s0)
            plsc.addupdate_scatter(strip, [i0 + 16], gx1 * s0)
            plsc.addupdate_scatter(strip, [i0 + 32], gx2 * s0)
            i1 = i0 + _SW
            plsc.addupdate_scatter(strip, [i1], gx0 * s1)
            plsc.addupdate_scatter(strip, [i1 + 16], gx1 * s1)
            plsc.addupdate_scatter(strip, [i1 + 32], gx2 * s1)
            return i1 + _SW
        i0f = lax.fori_loop(0, npairs, pair_body, i0_init)

        @pl.when(nrows % 2 == 1)
        def _tail():
            s = wgytab[pl.ds(sbase + 2 * npairs, 16)][0]
            plsc.addupdate_scatter(strip, [i0f], gx0 * s)
            plsc.addupdate_scatter(strip, [i0f + 16], gx1 * s)
            plsc.addupdate_scatter(strip, [i0f + 32], gx2 * s)
        return _
    lax.fori_loop(0, cnt, cell_body, None)

    pltpu.sync_copy(strip, out_hbm.at[pl.ds(wid * _STRIP_WORDS, _STRIP_WORDS)])


def _tc_body(my_ref, vr_ref, mx_ref, vc_ref, c_ref, out_ref):
    hs, np_ = out_ref.shape[1], my_ref.shape[2]
    ws = out_ref.shape[2]
    my = my_ref[0]
    hv_r = 0.5 / vr_ref[0]
    mx = mx_ref[0]
    hv_c = 0.5 / vc_ref[0]
    c = c_ref[0]
    ceff = jnp.where(c > _MIN_CONF, c, 0.0)

    y = lax.broadcasted_iota(jnp.int32, (hs, np_), 0).astype(jnp.float32)
    dy = y - my
    gyt = jnp.exp(-(dy * dy) * hv_r)

    x = lax.broadcasted_iota(jnp.int32, (np_, ws), 1).astype(jnp.float32)
    dx = x - mx
    cgx = ceff * jnp.exp(-(dx * dx) * hv_c)

    out_ref[0] = jnp.dot(gyt, cgx, preferred_element_type=jnp.float32)


def _tc_part(mean, variance, confidence):
    nb = mean.shape[0]
    m = mean.reshape(nb, _N, 2)
    pad = _NP - _N
    mx = jnp.pad(m[..., 0], ((0, 0), (0, pad)))
    my = jnp.pad(m[..., 1], ((0, 0), (0, pad)))
    v = jnp.pad(variance.reshape(nb, _N), ((0, 0), (0, pad)),
                constant_values=1.0)
    c = jnp.pad(confidence.reshape(nb, _N), ((0, 0), (0, pad)))

    row_spec = pl.BlockSpec((1, 1, _NP), lambda i: (i, 0, 0))
    col_spec = pl.BlockSpec((1, _NP, 1), lambda i: (i, 0, 0))
    return pl.pallas_call(
        _tc_body,
        grid=(nb,),
        in_specs=[row_spec, row_spec, col_spec, col_spec, col_spec],
        out_specs=pl.BlockSpec((1, _HS, _WS), lambda i: (i, 0, 0)),
        out_shape=jax.ShapeDtypeStruct((nb, _HS, _WS), jnp.float32),
    )(my[:, None, :], v[:, None, :], mx[:, :, None], v[:, :, None],
      c[:, :, None])


def _sc_part(mean, variance, confidence):
    nb = mean.shape[0]
    m = mean.reshape(nb, _N, 2)
    pad = _NP - _N
    mx = jnp.pad(m[..., 0], ((0, 0), (0, pad)), constant_values=112.0)
    my = jnp.pad(m[..., 1], ((0, 0), (0, pad)), constant_values=112.0)
    v = jnp.pad(variance.reshape(nb, _N), ((0, 0), (0, pad)),
                constant_values=1.0)
    c = jnp.pad(confidence.reshape(nb, _N), ((0, 0), (0, pad)))
    r = jnp.ceil(jnp.sqrt(2.0 * jnp.log(1.0 / 3e-4) * v))
    alive = c > _MIN_CONF
    jl0 = jnp.where(alive, jnp.maximum(0.0, float(_R) - r), 0.0)
    jh0 = jnp.where(alive, jnp.minimum(float(_WIN), float(_R) + r + 2.0), 0.0)
    params = jnp.stack([mx, my, v, c, jl0, jh0, jnp.zeros_like(mx)],
                       axis=1).reshape(nb, _NROW * _NP)

    mesh = plsc.VectorSubcoreMesh(core_axis_name="c", subcore_axis_name="s")
    run = functools.partial(
        pl.kernel,
        mesh=mesh,
        compiler_params=pltpu.CompilerParams(needs_layout_passes=False),
        out_type=jax.ShapeDtypeStruct((nb * _HS * _SW,), jnp.float32),
        scratch_types=[
            pltpu.VMEM((_NROW * _NP,), jnp.float32),
            pltpu.VMEM((_NP * _WIN,), jnp.float32),
            pltpu.VMEM((_NP * _WIN + 16,), jnp.float32),
            pltpu.VMEM((_NP + 16,), jnp.float32),
            pltpu.VMEM((_NP + 16,), jnp.float32),
            pltpu.VMEM((_NP + 16,), jnp.float32),
            pltpu.VMEM((_NP + 16,), jnp.float32),
            pltpu.VMEM((_NP + 16,), jnp.int32),
            pltpu.VMEM((_NP + 16,), jnp.int32),
            pltpu.VMEM((_NP + 16,), jnp.int32),
            pltpu.VMEM((_NP + 16,), jnp.int32),
            pltpu.VMEM((_STRIP_WORDS,), jnp.float32),
        ],
    )(_sc_body)
    padded = run(params)
    return padded.reshape(nb, _HS, _SW)[:, :, _XPAD:_XPAD + _WS]


def kernel(mean, variance, confidence):
    sc = _sc_part(mean[:_KSC], variance[:_KSC], confidence[:_KSC])
    tc = _tc_part(mean[_KSC:], variance[_KSC:], confidence[_KSC:])
    return jnp.concatenate([sc, tc], axis=0)
